# baseline (speedup 1.0000x reference)
import dataclasses
import functools

import jax
import jax.numpy as jnp
from jax import lax
from jax.experimental import pallas as pl
from jax.experimental.pallas import tpu as pltpu
from jax.experimental.pallas import tpu_sc as plsc

DP = 128
L = 16
NC, NS = 2, 16
NW = NC * NS
W = 128


def _build_sc_kernel(E):
    PW = E // NW
    assert PW * NW == E and PW % 8 == 0
    NT = (PW + W - 1) // W
    last_off = PW - W

    cp = pltpu.CompilerParams()
    if "needs_layout_passes" in pltpu.CompilerParams.__dataclass_fields__:
        cp = dataclasses.replace(cp, needs_layout_passes=False)

    @functools.partial(
        pl.kernel,
        compiler_params=cp,
        mesh=plsc.VectorSubcoreMesh(core_axis_name="c", subcore_axis_name="s"),
        out_type=jax.ShapeDtypeStruct((E,), jnp.float32),
        scratch_types=[
            pltpu.VMEM((PW,), jnp.int32),
            pltpu.VMEM((PW,), jnp.int32),
            pltpu.VMEM((W, DP), jnp.int32),
            pltpu.VMEM((W, DP), jnp.int32),
            pltpu.VMEM((W, DP), jnp.int32),
            pltpu.VMEM((W, DP), jnp.int32),
            pltpu.VMEM((PW,), jnp.float32),
            pltpu.SemaphoreType.DMA,
            pltpu.SemaphoreType.DMA,
        ],
    )
    def k(x_hbm, src_hbm, dst_hbm, out_hbm,
          sidx, didx, srows0, drows0, srows1, drows1, outv, sem0, sem1):
        wid = lax.axis_index("s") * NC + lax.axis_index("c")
        base = wid * PW

        pltpu.sync_copy(src_hbm.at[pl.ds(base, PW)], sidx)
        pltpu.sync_copy(dst_hbm.at[pl.ds(base, PW)], didx)

        def off(t):
            return jnp.minimum(t * W, last_off)

        def issue(t, sb, db, sem):
            o = off(t)
            pltpu.async_copy(x_hbm.at[sidx.at[pl.ds(o, W)]], sb, sem)
            pltpu.async_copy(x_hbm.at[didx.at[pl.ds(o, W)]], db, sem)

        def drain(sb, db, sem):
            pltpu.make_async_copy(x_hbm.at[sidx.at[pl.ds(0, W)]], sb, sem).wait()
            pltpu.make_async_copy(x_hbm.at[didx.at[pl.ds(0, W)]], db, sem).wait()

        def unpack2(v):
            lo = plsc.bitcast(lax.shift_left(v, 16), jnp.float32)
            hi = plsc.bitcast(v & jnp.int32(-65536), jnp.float32)
            return lo, hi

        def compute(t, sb, db):
            o = off(t)
            lane = lax.broadcasted_iota(jnp.int32, (L,), 0)
            zero = jnp.zeros((L,), jnp.float32)
            for g in range(0, W // L, 2):
                e0 = lane + (g * L)
                e1 = lane + ((g + 1) * L)

                def body(k, accs):
                    a0, a1 = accs
                    kv = (lane + k) & (DP - 1)
                    s0l, s0h = unpack2(plsc.load_gather(sb, [e0, kv]))
                    d0l, d0h = unpack2(plsc.load_gather(db, [e0, kv]))
                    s1l, s1h = unpack2(plsc.load_gather(sb, [e1, kv]))
                    d1l, d1h = unpack2(plsc.load_gather(db, [e1, kv]))
                    a0 = a0 + s0l * d0l + s0h * d0h
                    a1 = a1 + s1l * d1l + s1h * d1h
                    return (a0, a1)

                acc0, acc1 = lax.fori_loop(0, DP, body, (zero, zero),
                                           unroll=8)
                outv[pl.ds(o + g * L, L)] = acc0
                outv[pl.ds(o + (g + 1) * L, L)] = acc1

        issue(0, srows0, drows0, sem0)

        @pl.loop(0, NT, step=2)
        def _(t):
            @pl.when(t + 1 < NT)
            def _():
                issue(t + 1, srows1, drows1, sem1)

            drain(srows0, drows0, sem0)
            compute(t, srows0, drows0)

            @pl.when(t + 2 < NT)
            def _():
                issue(t + 2, srows0, drows0, sem0)

            @pl.when(t + 1 < NT)
            def _():
                drain(srows1, drows1, sem1)
                compute(t + 1, srows1, drows1)

        pltpu.sync_copy(outv, out_hbm.at[pl.ds(base, PW)])

    return k


@jax.jit
def kernel(x, edge_index):
    src = edge_index[0].astype(jnp.int32)
    dst = edge_index[1].astype(jnp.int32)
    x_pairs = lax.bitcast_convert_type(
        x.astype(jnp.bfloat16).reshape(x.shape[0], x.shape[1] // 2, 2),
        jnp.int32)
    return _build_sc_kernel(src.shape[0])(x_pairs, src, dst)


# speedup vs baseline: 1.4640x; 1.4640x over previous
import dataclasses
import functools

import jax
import jax.numpy as jnp
from jax import lax
from jax.experimental import pallas as pl
from jax.experimental.pallas import tpu as pltpu
from jax.experimental.pallas import tpu_sc as plsc

D = 256
L = 16
NC, NS = 2, 16
NW = NC * NS
W = 64


def _lane_shuffle(v, idx):
    dnums = lax.GatherDimensionNumbers(
        offset_dims=(), collapsed_slice_dims=(0,), start_index_map=(0,))
    return lax.gather(v, idx[:, None], dnums, (1,),
                      mode=lax.GatherScatterMode.PROMISE_IN_BOUNDS)


def _build_sc_kernel(E):
    PW = E // NW
    assert PW * NW == E and PW % 8 == 0
    NT = (PW + W - 1) // W
    last_off = PW - W

    cp = pltpu.CompilerParams()
    if "needs_layout_passes" in pltpu.CompilerParams.__dataclass_fields__:
        cp = dataclasses.replace(cp, needs_layout_passes=False)

    @functools.partial(
        pl.kernel,
        compiler_params=cp,
        mesh=plsc.VectorSubcoreMesh(core_axis_name="c", subcore_axis_name="s"),
        out_type=jax.ShapeDtypeStruct((E,), jnp.float32),
        scratch_types=[
            pltpu.VMEM((PW,), jnp.int32),
            pltpu.VMEM((PW,), jnp.int32),
            pltpu.VMEM((W, D), jnp.float32),
            pltpu.VMEM((W, D), jnp.float32),
            pltpu.VMEM((W, D), jnp.float32),
            pltpu.VMEM((W, D), jnp.float32),
            pltpu.VMEM((W, D), jnp.float32),
            pltpu.VMEM((W, D), jnp.float32),
            pltpu.VMEM((PW,), jnp.float32),
            pltpu.SemaphoreType.DMA,
            pltpu.SemaphoreType.DMA,
            pltpu.SemaphoreType.DMA,
        ],
    )
    def k(x_hbm, src_hbm, dst_hbm, out_hbm,
          sidx, didx, srows0, drows0, srows1, drows1, srows2, drows2,
          outv, sem0, sem1, sem2):
        wid = lax.axis_index("s") * NC + lax.axis_index("c")
        base = wid * PW

        pltpu.sync_copy(src_hbm.at[pl.ds(base, PW)], sidx)
        pltpu.sync_copy(dst_hbm.at[pl.ds(base, PW)], didx)

        def off(t):
            return jnp.minimum(t * W, last_off)

        def issue(t, sb, db, sem):
            o = off(t)
            pltpu.async_copy(x_hbm.at[sidx.at[pl.ds(o, W)]], sb, sem)
            pltpu.async_copy(x_hbm.at[didx.at[pl.ds(o, W)]], db, sem)

        def drain(sb, db, sem):
            pltpu.make_async_copy(x_hbm.at[sidx.at[pl.ds(0, W)]], sb, sem).wait()
            pltpu.make_async_copy(x_hbm.at[didx.at[pl.ds(0, W)]], db, sem).wait()

        def compute(t, sb, db):
            o = off(t)
            lane = lax.broadcasted_iota(jnp.int32, (L,), 0)
            zero = jnp.zeros((L,), jnp.float32)
            for g in range(0, W // L, 2):
                e0 = lane + (g * L)
                e1 = lane + ((g + 1) * L)

                def body(k, accs):
                    a0, a1 = accs
                    kv = (lane + k) & (D - 1)
                    sv0 = plsc.load_gather(sb, [e0, kv])
                    dv0 = plsc.load_gather(db, [e0, kv])
                    sv1 = plsc.load_gather(sb, [e1, kv])
                    dv1 = plsc.load_gather(db, [e1, kv])
                    return (a0 + sv0 * dv0, a1 + sv1 * dv1)

                acc0, acc1 = lax.fori_loop(0, D, body, (zero, zero),
                                           unroll=8)
                outv[pl.ds(o + g * L, L)] = acc0
                outv[pl.ds(o + (g + 1) * L, L)] = acc1

        issue(0, srows0, drows0, sem0)
        issue(1, srows1, drows1, sem1)

        @pl.loop(0, NT, step=3)
        def _(t):
            @pl.when(t + 2 < NT)
            def _():
                issue(t + 2, srows2, drows2, sem2)

            drain(srows0, drows0, sem0)
            compute(t, srows0, drows0)

            @pl.when(t + 3 < NT)
            def _():
                issue(t + 3, srows0, drows0, sem0)

            @pl.when(t + 1 < NT)
            def _():
                drain(srows1, drows1, sem1)
                compute(t + 1, srows1, drows1)

            @pl.when(t + 4 < NT)
            def _():
                issue(t + 4, srows1, drows1, sem1)

            @pl.when(t + 2 < NT)
            def _():
                drain(srows2, drows2, sem2)
                compute(t + 2, srows2, drows2)

        pltpu.sync_copy(outv, out_hbm.at[pl.ds(base, PW)])

    return k


@jax.jit
def kernel(x, edge_index):
    src = edge_index[0].astype(jnp.int32)
    dst = edge_index[1].astype(jnp.int32)
    return _build_sc_kernel(src.shape[0])(x, src, dst)


# speedup vs baseline: 1.4655x; 1.0010x over previous
import dataclasses
import functools

import jax
import jax.numpy as jnp
from jax import lax
from jax.experimental import pallas as pl
from jax.experimental.pallas import tpu as pltpu
from jax.experimental.pallas import tpu_sc as plsc

D = 256
L = 16
NC, NS = 2, 16
NW = NC * NS
W = 48


def _lane_shuffle(v, idx):
    dnums = lax.GatherDimensionNumbers(
        offset_dims=(), collapsed_slice_dims=(0,), start_index_map=(0,))
    return lax.gather(v, idx[:, None], dnums, (1,),
                      mode=lax.GatherScatterMode.PROMISE_IN_BOUNDS)


def _build_sc_kernel(E):
    PW = E // NW
    assert PW * NW == E and PW % 8 == 0
    NT = (PW + W - 1) // W
    last_off = PW - W

    cp = pltpu.CompilerParams()
    if "needs_layout_passes" in pltpu.CompilerParams.__dataclass_fields__:
        cp = dataclasses.replace(cp, needs_layout_passes=False)

    @functools.partial(
        pl.kernel,
        compiler_params=cp,
        mesh=plsc.VectorSubcoreMesh(core_axis_name="c", subcore_axis_name="s"),
        out_type=jax.ShapeDtypeStruct((E,), jnp.float32),
        scratch_types=[
            pltpu.VMEM((PW,), jnp.int32),
            pltpu.VMEM((PW,), jnp.int32),
            pltpu.VMEM((W, D), jnp.float32),
            pltpu.VMEM((W, D), jnp.float32),
            pltpu.VMEM((W, D), jnp.float32),
            pltpu.VMEM((W, D), jnp.float32),
            pltpu.VMEM((W, D), jnp.float32),
            pltpu.VMEM((W, D), jnp.float32),
            pltpu.VMEM((W, D), jnp.float32),
            pltpu.VMEM((W, D), jnp.float32),
            pltpu.VMEM((PW,), jnp.float32),
            pltpu.SemaphoreType.DMA,
            pltpu.SemaphoreType.DMA,
            pltpu.SemaphoreType.DMA,
            pltpu.SemaphoreType.DMA,
        ],
    )
    def k(x_hbm, src_hbm, dst_hbm, out_hbm,
          sidx, didx, srows0, drows0, srows1, drows1, srows2, drows2,
          srows3, drows3, outv, sem0, sem1, sem2, sem3):
        wid = lax.axis_index("s") * NC + lax.axis_index("c")
        base = wid * PW

        pltpu.sync_copy(src_hbm.at[pl.ds(base, PW)], sidx)
        pltpu.sync_copy(dst_hbm.at[pl.ds(base, PW)], didx)

        def off(t):
            return jnp.minimum(t * W, last_off)

        def issue(t, sb, db, sem):
            o = off(t)
            pltpu.async_copy(x_hbm.at[sidx.at[pl.ds(o, W)]], sb, sem)
            pltpu.async_copy(x_hbm.at[didx.at[pl.ds(o, W)]], db, sem)

        def drain(sb, db, sem):
            pltpu.make_async_copy(x_hbm.at[sidx.at[pl.ds(0, W)]], sb, sem).wait()
            pltpu.make_async_copy(x_hbm.at[didx.at[pl.ds(0, W)]], db, sem).wait()

        def compute(t, sb, db):
            o = off(t)
            lane = lax.broadcasted_iota(jnp.int32, (L,), 0)
            zero = jnp.zeros((L,), jnp.float32)
            ng = W // L
            for g in range(0, ng - 1, 2):
                e0 = lane + (g * L)
                e1 = lane + ((g + 1) * L)

                def body(k, accs):
                    a0, a1 = accs
                    kv = (lane + k) & (D - 1)
                    sv0 = plsc.load_gather(sb, [e0, kv])
                    dv0 = plsc.load_gather(db, [e0, kv])
                    sv1 = plsc.load_gather(sb, [e1, kv])
                    dv1 = plsc.load_gather(db, [e1, kv])
                    return (a0 + sv0 * dv0, a1 + sv1 * dv1)

                acc0, acc1 = lax.fori_loop(0, D, body, (zero, zero),
                                           unroll=8)
                outv[pl.ds(o + g * L, L)] = acc0
                outv[pl.ds(o + (g + 1) * L, L)] = acc1
            if ng % 2:
                g = ng - 1
                e0 = lane + (g * L)

                def body1(k, acc):
                    kv = (lane + k) & (D - 1)
                    sv = plsc.load_gather(sb, [e0, kv])
                    dv = plsc.load_gather(db, [e0, kv])
                    return acc + sv * dv

                acc = lax.fori_loop(0, D, body1, zero, unroll=8)
                outv[pl.ds(o + g * L, L)] = acc

        issue(0, srows0, drows0, sem0)
        issue(1, srows1, drows1, sem1)
        issue(2, srows2, drows2, sem2)

        bufs = ((srows0, drows0, sem0), (srows1, drows1, sem1),
                (srows2, drows2, sem2), (srows3, drows3, sem3))

        @pl.loop(0, NT, step=4)
        def _(t):
            for i in range(4):
                @pl.when(t + i + 3 < NT)
                def _():
                    sb, db, sem = bufs[(i + 3) % 4]
                    issue(t + i + 3, sb, db, sem)

                @pl.when(t + i < NT)
                def _():
                    sb, db, sem = bufs[i]
                    drain(sb, db, sem)
                    compute(t + i, sb, db)

        pltpu.sync_copy(outv, out_hbm.at[pl.ds(base, PW)])

    return k


@jax.jit
def kernel(x, edge_index):
    src = edge_index[0].astype(jnp.int32)
    dst = edge_index[1].astype(jnp.int32)
    return _build_sc_kernel(src.shape[0])(x, src, dst)
